# baseline (speedup 1.0000x reference)
import jax
import jax.numpy as jnp
from jax import lax
from jax.experimental import pallas as pl
from jax.experimental.pallas import tpu as pltpu
from jax.experimental.pallas import tpu_sc as plsc

N_TABLE = 50
N_TABLE_PAD = 64
NW = 32
LANES = 16
BLK_R = 8
BLK_C = 1024
NBUF = 4


def _sc_body(*refs):
    species_hbm, volume_hbm, pf_hbm, vf_hbm, out_hbm = refs[:5]
    ratio_v, pf_v, vf_v = refs[5:8]
    sp_b = list(refs[8:8 + NBUF])
    vol_b = list(refs[8 + NBUF:8 + 2 * NBUF])
    out_b = list(refs[8 + 2 * NBUF:8 + 3 * NBUF])
    sp_sems = list(refs[8 + 3 * NBUF:8 + 4 * NBUF])
    vol_sems = list(refs[8 + 4 * NBUF:8 + 5 * NBUF])
    out_sems = list(refs[8 + 5 * NBUF:8 + 6 * NBUF])

    rows, cols = out_hbm.shape
    n_blocks = (rows // BLK_R) * (cols // BLK_C)
    per_w = n_blocks // NW
    wid = lax.axis_index("s") * 2 + lax.axis_index("c")

    pltpu.sync_copy(pf_hbm, pf_v.at[pl.ds(0, N_TABLE)])
    pltpu.sync_copy(vf_hbm, vf_v.at[pl.ds(0, N_TABLE)])
    for j in range(N_TABLE_PAD // LANES):
        sl = pl.ds(j * LANES, LANES)
        ratio_v[sl] = pf_v[sl] / vf_v[sl]

    halves = cols // BLK_C

    def block_slices(g):
        blk = wid * per_w + g
        r0 = (blk // halves) * BLK_R
        c0 = (blk % halves) * BLK_C
        return pl.ds(r0, BLK_R), pl.ds(c0, BLK_C)

    def start_in(g, b):
        rs, cs = block_slices(g)
        pltpu.async_copy(species_hbm.at[rs, cs], sp_b[b], sp_sems[b])
        pltpu.async_copy(volume_hbm.at[rs, cs], vol_b[b], vol_sems[b])

    def wait_in(g, b):
        rs, cs = block_slices(g)
        pltpu.make_async_copy(species_hbm.at[rs, cs], sp_b[b], sp_sems[b]).wait()
        pltpu.make_async_copy(volume_hbm.at[rs, cs], vol_b[b], vol_sems[b]).wait()

    def start_out(g, b):
        rs, cs = block_slices(g)
        pltpu.async_copy(out_b[b], out_hbm.at[rs, cs], out_sems[b])

    def wait_out(g, b):
        rs, cs = block_slices(g)
        pltpu.make_async_copy(out_b[b], out_hbm.at[rs, cs], out_sems[b]).wait()

    def compute(b):
        spb, volb, outb = sp_b[b], vol_b[b], out_b[b]
        for r in range(BLK_R):
            @plsc.parallel_loop(0, BLK_C, step=LANES, unroll=8)
            def _(i):
                sl = pl.ds(i, LANES)
                rv = plsc.load_gather(ratio_v, [spb[r, sl]])
                outb[r, sl] = volb[r, sl] * rv

    npairs = per_w // NBUF

    for g in range(NBUF):
        start_in(g, g)
    for b in range(NBUF):
        wait_in(b, b)
        compute(b)
        start_out(b, b)
        start_in(b + NBUF, b)

    def pair_body(gg, carry):
        for b in range(NBUF):
            g = gg * NBUF + b
            wait_in(g, b)
            wait_out(g - NBUF, b)
            compute(b)
            start_out(g, b)
            start_in(g + NBUF, b)
        return carry

    lax.fori_loop(1, npairs - 1, pair_body, 0)

    for b in range(NBUF):
        g = per_w - NBUF + b
        wait_in(g, b)
        wait_out(g - NBUF, b)
        compute(b)
        start_out(g, b)
    for b in range(NBUF):
        wait_out(per_w - NBUF + b, b)


def kernel(species, volume, polar_free, volume_free):
    mesh = plsc.VectorSubcoreMesh(core_axis_name="c", subcore_axis_name="s")
    run = pl.kernel(
        _sc_body,
        out_type=jax.ShapeDtypeStruct(species.shape, jnp.float32),
        mesh=mesh,
        scratch_types=[
            pltpu.VMEM((N_TABLE_PAD,), jnp.float32),
            pltpu.VMEM((N_TABLE_PAD,), jnp.float32),
            pltpu.VMEM((N_TABLE_PAD,), jnp.float32),
            *[pltpu.VMEM((BLK_R, BLK_C), jnp.int32) for _ in range(NBUF)],
            *[pltpu.VMEM((BLK_R, BLK_C), jnp.float32) for _ in range(NBUF)],
            *[pltpu.VMEM((BLK_R, BLK_C), jnp.float32) for _ in range(NBUF)],
            *[pltpu.SemaphoreType.DMA for _ in range(3 * NBUF)],
        ],
        compiler_params=pltpu.CompilerParams(needs_layout_passes=False,
                                             use_tc_tiling_on_sc=True),
    )
    return run(species.astype(jnp.int32), volume, polar_free, volume_free)


# speedup vs baseline: 1.1508x; 1.1508x over previous
import jax
import jax.numpy as jnp
from jax import lax
from jax.experimental import pallas as pl
from jax.experimental.pallas import tpu as pltpu
from jax.experimental.pallas import tpu_sc as plsc

N_TABLE = 50
N_TABLE_PAD = 64
NW = 32
LANES = 16
BLK_R = 8
BLK_C = 2048
NBUF = 2


def _sc_body(*refs):
    species_hbm, volume_hbm, pf_hbm, vf_hbm, out_hbm = refs[:5]
    ratio_v, pf_v, vf_v = refs[5:8]
    sp_b = list(refs[8:8 + NBUF])
    vol_b = list(refs[8 + NBUF:8 + 2 * NBUF])
    out_b = list(refs[8 + 2 * NBUF:8 + 3 * NBUF])
    sp_sems = list(refs[8 + 3 * NBUF:8 + 4 * NBUF])
    vol_sems = list(refs[8 + 4 * NBUF:8 + 5 * NBUF])
    out_sems = list(refs[8 + 5 * NBUF:8 + 6 * NBUF])

    rows, cols = out_hbm.shape
    n_blocks = (rows // BLK_R) * (cols // BLK_C)
    per_w = n_blocks // NW
    wid = lax.axis_index("s") * 2 + lax.axis_index("c")

    pltpu.sync_copy(pf_hbm, pf_v.at[pl.ds(0, N_TABLE)])
    pltpu.sync_copy(vf_hbm, vf_v.at[pl.ds(0, N_TABLE)])
    for j in range(N_TABLE_PAD // LANES):
        sl = pl.ds(j * LANES, LANES)
        ratio_v[sl] = pf_v[sl] / vf_v[sl]

    halves = cols // BLK_C

    def block_slices(g):
        blk = wid * per_w + g
        r0 = (blk // halves) * BLK_R
        c0 = (blk % halves) * BLK_C
        return pl.ds(r0, BLK_R), pl.ds(c0, BLK_C)

    def start_in(g, b):
        rs, cs = block_slices(g)
        pltpu.async_copy(species_hbm.at[rs, cs], sp_b[b], sp_sems[b])
        pltpu.async_copy(volume_hbm.at[rs, cs], vol_b[b], vol_sems[b])

    def wait_in(g, b):
        rs, cs = block_slices(g)
        pltpu.make_async_copy(species_hbm.at[rs, cs], sp_b[b], sp_sems[b]).wait()
        pltpu.make_async_copy(volume_hbm.at[rs, cs], vol_b[b], vol_sems[b]).wait()

    def start_out(g, b):
        rs, cs = block_slices(g)
        pltpu.async_copy(out_b[b], out_hbm.at[rs, cs], out_sems[b])

    def wait_out(g, b):
        rs, cs = block_slices(g)
        pltpu.make_async_copy(out_b[b], out_hbm.at[rs, cs], out_sems[b]).wait()

    def compute(b):
        spb, volb, outb = sp_b[b], vol_b[b], out_b[b]

        @plsc.parallel_loop(0, BLK_R * BLK_C, step=LANES, unroll=8)
        def _(i):
            r = lax.shift_right_logical(i, 11)
            sl = pl.ds(lax.bitwise_and(i, BLK_C - 1), LANES)
            rv = plsc.load_gather(ratio_v, [spb[r, sl]])
            outb[r, sl] = volb[r, sl] * rv

    npairs = per_w // NBUF

    for g in range(NBUF):
        start_in(g, g)
    for b in range(NBUF):
        wait_in(b, b)
        compute(b)
        start_out(b, b)
        start_in(b + NBUF, b)

    def pair_body(gg, carry):
        for b in range(NBUF):
            g = gg * NBUF + b
            wait_in(g, b)
            wait_out(g - NBUF, b)
            compute(b)
            start_out(g, b)
            start_in(g + NBUF, b)
        return carry

    lax.fori_loop(1, npairs - 1, pair_body, 0)

    for b in range(NBUF):
        g = per_w - NBUF + b
        wait_in(g, b)
        wait_out(g - NBUF, b)
        compute(b)
        start_out(g, b)
    for b in range(NBUF):
        wait_out(per_w - NBUF + b, b)


def kernel(species, volume, polar_free, volume_free):
    mesh = plsc.VectorSubcoreMesh(core_axis_name="c", subcore_axis_name="s")
    run = pl.kernel(
        _sc_body,
        out_type=jax.ShapeDtypeStruct(species.shape, jnp.float32),
        mesh=mesh,
        scratch_types=[
            pltpu.VMEM((N_TABLE_PAD,), jnp.float32),
            pltpu.VMEM((N_TABLE_PAD,), jnp.float32),
            pltpu.VMEM((N_TABLE_PAD,), jnp.float32),
            *[pltpu.VMEM((BLK_R, BLK_C), jnp.int32) for _ in range(NBUF)],
            *[pltpu.VMEM((BLK_R, BLK_C), jnp.float32) for _ in range(NBUF)],
            *[pltpu.VMEM((BLK_R, BLK_C), jnp.float32) for _ in range(NBUF)],
            *[pltpu.SemaphoreType.DMA for _ in range(3 * NBUF)],
        ],
        compiler_params=pltpu.CompilerParams(needs_layout_passes=False,
                                             use_tc_tiling_on_sc=True),
    )
    return run(species.astype(jnp.int32), volume, polar_free, volume_free)
